# baseline (speedup 1.0000x reference)
import functools

import jax
import jax.numpy as jnp
from jax import lax
from jax.experimental import pallas as pl
from jax.experimental.pallas import tpu as pltpu
from jax.experimental.pallas import tpu_sc as plsc

_N = 10000
_D = 128
_E = 320000
_EPS = 1e-5

_NC = 2
_NS = 16
_NW = _NC * _NS
_EPT = _E // _NW
_CH = 125
_NCHUNK = _EPT // _CH
_NPASS = 10
_CPP = _NCHUNK // _NPASS
_RPT = 624
_TAIL0 = _RPT * _NS
_TAILN = _N - _TAIL0

_mesh = plsc.VectorSubcoreMesh(core_axis_name="c", subcore_axis_name="s")


def _init_zero(z_hbm, dst_s, s):
  row0 = s * _RPT
  pltpu.sync_copy(z_hbm.at[pl.ds(row0, _RPT)], dst_s.at[pl.ds(row0, _RPT)])

  @pl.when(s == _NS - 1)
  def _():
    pltpu.sync_copy(z_hbm.at[pl.ds(_TAIL0, _TAILN)],
                    dst_s.at[pl.ds(_TAIL0, _TAILN)])


def _write_out(src_s, out_hbm, c, s):
  row0 = s * _RPT
  pltpu.sync_copy(src_s.at[pl.ds(row0, _RPT)],
                  out_hbm.at[c, pl.ds(row0, _RPT)])

  @pl.when(s == _NS - 1)
  def _():
    pltpu.sync_copy(src_s.at[pl.ds(_TAIL0, _TAILN)],
                    out_hbm.at[c, pl.ds(_TAIL0, _TAILN)])


@functools.partial(
    pl.kernel,
    out_type=(jax.ShapeDtypeStruct((_NC, _N, _D), jnp.float32),),
    mesh=_mesh,
    scratch_types=[
        pltpu.VMEM((2 * _CPP, _CH), jnp.int32),
        pltpu.VMEM((2 * _CPP, _CH), jnp.int32),
        pltpu.VMEM((_CH, _D), jnp.float32),
        pltpu.VMEM((_CH, _D), jnp.float32),
        pltpu.VMEM_SHARED((_N, _D), jnp.float32),
        pltpu.SemaphoreType.DMA,
        pltpu.SemaphoreType.DMA,
        pltpu.SemaphoreType.DMA,
        pltpu.SemaphoreType.DMA,
        pltpu.SemaphoreType.DMA,
        pltpu.SemaphoreType.DMA,
    ],
)
def _sc_agg(ei_hbm, x_hbm, z_hbm, zrows_hbm, agg_out,
            idx_a, idx_b, rows_a, rows_b, agg_s,
            sem_ga, sem_gb, sem_sa, sem_sb, sem_ia, sem_ib):
  c = lax.axis_index("c")
  s = lax.axis_index("s")
  wid = c * _NS + s
  _init_zero(z_hbm, agg_s, s)

  rows = (rows_a, rows_b)
  gsem = (sem_ga, sem_gb)
  ssem = (sem_sa, sem_sb)
  idxs = (idx_a, idx_b)

  def issue_g(b, idx_ref, r):
    return pltpu.async_copy(x_hbm.at[idx_ref.at[2 * r]], rows[b], gsem[b])

  def issue_s(b, idx_ref, r):
    return pltpu.async_copy(rows[b], agg_s.at[idx_ref.at[2 * r + 1]],
                            ssem[b], add=True)

  def wait_g(b):
    pltpu.make_async_copy(x_hbm.at[idx_a.at[0]], rows[b], gsem[b]).wait()

  def wait_s(b):
    pltpu.make_async_copy(rows[b], agg_s.at[idx_a.at[1]], ssem[b]).wait()

  pltpu.sync_copy(ei_hbm.at[wid, pl.ds(0, 2 * _CPP)], idx_a)
  pltpu.sync_copy(zrows_hbm, rows_b)
  issue_g(0, idx_a, 0)
  plsc.subcore_barrier()
  issue_s(1, idx_a, 0)

  def body(q, carry):
    pf = {}
    for k in range(16):
      b = k % 2
      o = 1 - b
      sseg = idxs[0] if k < 8 else idxs[1]
      gk = k + 1
      gseg = idxs[0] if gk < 8 or gk == 16 else idxs[1]
      wait_g(b)
      issue_s(b, sseg, k % 8)
      wait_s(o)
      if k == 0:
        off_b = pl.multiple_of((2 * q + 1) * (2 * _CPP), 2 * _CPP)
        pf["b"] = pltpu.async_copy(
            ei_hbm.at[wid, pl.ds(off_b, 2 * _CPP)], idx_b, sem_ib)
      if k == 8:
        off_a = pl.multiple_of(
            jnp.minimum(2 * q + 2, _NPASS - 1) * (2 * _CPP), 2 * _CPP)
        pf["a"] = pltpu.async_copy(
            ei_hbm.at[wid, pl.ds(off_a, 2 * _CPP)], idx_a, sem_ia)
      if k == 6:
        pf["b"].wait()
      if k == 14:
        pf["a"].wait()
      issue_g(o, gseg, gk % 8)
    return carry

  lax.fori_loop(0, _NPASS // 2, body, 0)
  wait_g(0)
  wait_s(1)
  plsc.subcore_barrier()
  _write_out(agg_s, agg_out, c, s)


@functools.partial(
    pl.kernel,
    out_type=(jax.ShapeDtypeStruct((_NC, _N, _D), jnp.float32),),
    mesh=_mesh,
    scratch_types=[
        pltpu.VMEM((_CPP, _CH), jnp.int32),
        pltpu.VMEM((_CPP, _CH), jnp.int32),
        pltpu.VMEM((_CH, _D), jnp.float32),
        pltpu.VMEM_SHARED((_N, _D), jnp.float32),
        pltpu.SemaphoreType.DMA,
        pltpu.SemaphoreType.DMA,
        pltpu.SemaphoreType.DMA,
    ],
)
def _sc_deg(dst_hbm, z_hbm, ones_hbm, deg_out,
            dst_a, dst_b, ones_v, deg_s, sem, sem_ia, sem_ib):
  c = lax.axis_index("c")
  s = lax.axis_index("s")
  wid = c * _NS + s
  _init_zero(z_hbm, deg_s, s)
  pltpu.sync_copy(ones_hbm, ones_v)

  def burst(dst_ref):
    sd = [pltpu.async_copy(ones_v, deg_s.at[dst_ref.at[j]], sem, add=True)
          for j in range(_CPP)]
    for d in sd:
      d.wait()

  pltpu.sync_copy(dst_hbm.at[wid, pl.ds(0, _CPP)], dst_a)
  plsc.subcore_barrier()

  def dbl_pass(q, carry):
    off_b = pl.multiple_of((2 * q + 1) * _CPP, _CPP)
    pfb = pltpu.async_copy(dst_hbm.at[wid, pl.ds(off_b, _CPP)], dst_b, sem_ib)
    burst(dst_a)
    pfb.wait()
    off_a = pl.multiple_of(
        jnp.minimum(2 * q + 2, _NPASS - 1) * _CPP, _CPP)
    pfa = pltpu.async_copy(dst_hbm.at[wid, pl.ds(off_a, _CPP)], dst_a, sem_ia)
    burst(dst_b)
    pfa.wait()
    return carry

  lax.fori_loop(0, _NPASS // 2, dbl_pass, 0)
  plsc.subcore_barrier()
  _write_out(deg_s, deg_out, c, s)


_NB = 5
_BR = _N // _NB


def _mm2(a, wa, b, wb, bias):
  return (jnp.dot(a, wa, preferred_element_type=jnp.float32,
                  precision=lax.Precision.HIGHEST)
          + jnp.dot(b, wb, preferred_element_type=jnp.float32,
                    precision=lax.Precision.HIGHEST)
          + bias)


def _mean_agg(agg_ref, deg_ref):
  agg = agg_ref[0] + agg_ref[1]
  deg = deg_ref[0][:, 0] + deg_ref[1][:, 0]
  return agg / jnp.clip(deg, 1.0, None)[:, None]


def _tc1_body(agg_ref, deg_ref, x_ref, wl_ref, wr_ref, b_ref, g_ref,
              beta_ref, h_ref, hpre_s, sum_s, sq_s):
  i = pl.program_id(0)

  @pl.when(i < _NB)
  def _():
    mean = _mean_agg(agg_ref, deg_ref)
    h = _mm2(mean, wl_ref[...], x_ref[...], wr_ref[...], b_ref[...])
    hpre_s[i] = h

    @pl.when(i == 0)
    def _():
      sum_s[...] = jnp.zeros_like(sum_s)
      sq_s[...] = jnp.zeros_like(sq_s)

    sum_s[0, :] += jnp.sum(h, axis=0)
    sq_s[0, :] += jnp.sum(jnp.square(h), axis=0)

  @pl.when(i >= _NB)
  def _():
    mu = sum_s[0, :] * (1.0 / _N)
    var = sq_s[0, :] * (1.0 / _N) - jnp.square(mu)
    scale = g_ref[...] * lax.rsqrt(var + _EPS)
    hp = hpre_s[i - _NB]
    h_ref[...] = jnp.maximum((hp - mu) * scale + beta_ref[...], 0.0)


def _tc2_body(agg_ref, deg_ref, h_ref, wl_ref, wr_ref, b_ref, out_ref):
  mean = _mean_agg(agg_ref, deg_ref)
  out_ref[...] = _mm2(mean, wl_ref[...], h_ref[...], wr_ref[...], b_ref[...])


def _phase1_blk(i):
  return (jnp.where(i < _NB, i, 0), 0)


_blk_agg1 = pl.BlockSpec((_NC, _BR, _D), lambda i: (0,) + _phase1_blk(i))
_blk_row1 = pl.BlockSpec((_BR, _D), _phase1_blk)
_blk_vec1 = pl.BlockSpec((_D,), lambda i: (0,))
_blk_mat1 = pl.BlockSpec((_D, _D), lambda i: (0, 0))
_blk_out1 = pl.BlockSpec((_BR, _D), lambda i: (jnp.where(i < _NB, 0, i - _NB), 0))

_tc1 = pl.pallas_call(
    _tc1_body,
    grid=(2 * _NB,),
    in_specs=[_blk_agg1, _blk_agg1, _blk_row1, _blk_mat1, _blk_mat1,
              _blk_vec1, _blk_vec1, _blk_vec1],
    out_specs=_blk_out1,
    out_shape=jax.ShapeDtypeStruct((_N, _D), jnp.float32),
    scratch_shapes=[pltpu.VMEM((_NB, _BR, _D), jnp.float32),
                    pltpu.VMEM((1, _D), jnp.float32),
                    pltpu.VMEM((1, _D), jnp.float32)])

_blk_agg = pl.BlockSpec((_NC, _BR, _D), lambda i: (0, i, 0))
_blk_row = pl.BlockSpec((_BR, _D), lambda i: (i, 0))
_blk_vec = pl.BlockSpec((_D,), lambda i: (0,))
_blk_mat = pl.BlockSpec((_D, _D), lambda i: (0, 0))

_tc2 = pl.pallas_call(
    _tc2_body,
    grid=(_NB,),
    in_specs=[_blk_agg, _blk_agg, _blk_row, _blk_mat, _blk_mat, _blk_vec],
    out_specs=_blk_row,
    out_shape=jax.ShapeDtypeStruct((_N, _D), jnp.float32))


def kernel(x, edge_index, W1_l, W1_r, b1, gamma, beta, W2_l, W2_r, b2):
  src = edge_index[0].astype(jnp.int32).reshape(_NW, _NCHUNK, _CH)
  dst = edge_index[1].astype(jnp.int32).reshape(_NW, _NCHUNK, _CH)
  ei = jnp.stack([src, dst], axis=2).reshape(_NW, 2 * _NCHUNK, _CH)
  z = jnp.zeros((_N, _D), jnp.float32)
  zrows = jnp.zeros((_CH, _D), jnp.float32)
  ones = jnp.ones((_CH, _D), jnp.float32)

  (deg,) = _sc_deg(dst, z, ones)
  (agg1,) = _sc_agg(ei, x, z, zrows)
  h = _tc1(agg1, deg, x, W1_l, W1_r, b1, gamma, beta)
  (agg2,) = _sc_agg(ei, h, z, zrows)
  out = _tc2(agg2, deg, h, W2_l, W2_r, b2)
  return out


# speedup vs baseline: 1.0649x; 1.0649x over previous
import functools

import jax
import jax.numpy as jnp
from jax import lax
from jax.experimental import pallas as pl
from jax.experimental.pallas import tpu as pltpu
from jax.experimental.pallas import tpu_sc as plsc

_N = 10000
_D = 128
_E = 320000
_EPS = 1e-5

_NC = 2
_NS = 16
_NW = _NC * _NS
_EPT = _E // _NW
_CH = 125
_NCHUNK = _EPT // _CH
_NPASS = 10
_CPP = _NCHUNK // _NPASS
_RPT = 624
_TAIL0 = _RPT * _NS
_TAILN = _N - _TAIL0

_mesh = plsc.VectorSubcoreMesh(core_axis_name="c", subcore_axis_name="s")


def _init_zero(z_hbm, dst_s, s):
  row0 = s * _RPT
  pltpu.sync_copy(z_hbm.at[pl.ds(row0, _RPT)], dst_s.at[pl.ds(row0, _RPT)])

  @pl.when(s == _NS - 1)
  def _():
    pltpu.sync_copy(z_hbm.at[pl.ds(_TAIL0, _TAILN)],
                    dst_s.at[pl.ds(_TAIL0, _TAILN)])


def _write_out(src_s, out_hbm, c, s):
  row0 = s * _RPT
  pltpu.sync_copy(src_s.at[pl.ds(row0, _RPT)],
                  out_hbm.at[c, pl.ds(row0, _RPT)])

  @pl.when(s == _NS - 1)
  def _():
    pltpu.sync_copy(src_s.at[pl.ds(_TAIL0, _TAILN)],
                    out_hbm.at[c, pl.ds(_TAIL0, _TAILN)])


@functools.partial(
    pl.kernel,
    out_type=(jax.ShapeDtypeStruct((_NC, _N, _D), jnp.float32),),
    mesh=_mesh,
    scratch_types=[
        pltpu.VMEM((2 * _CPP, _CH), jnp.int32),
        pltpu.VMEM((2 * _CPP, _CH), jnp.int32),
        pltpu.VMEM((_CH, _D), jnp.float32),
        pltpu.VMEM((_CH, _D), jnp.float32),
        pltpu.VMEM_SHARED((_N, _D), jnp.float32),
        pltpu.SemaphoreType.DMA,
        pltpu.SemaphoreType.DMA,
        pltpu.SemaphoreType.DMA,
        pltpu.SemaphoreType.DMA,
        pltpu.SemaphoreType.DMA,
        pltpu.SemaphoreType.DMA,
    ],
)
def _sc_agg(ei_hbm, x_hbm, z_hbm, agg_out,
            idx_a, idx_b, rows_a, rows_b, agg_s,
            sem_ga, sem_gb, sem_sa, sem_sb, sem_ia, sem_ib):
  c = lax.axis_index("c")
  s = lax.axis_index("s")
  wid = c * _NS + s
  _init_zero(z_hbm, agg_s, s)

  bufs = ((rows_a, sem_ga, sem_sa), (rows_b, sem_gb, sem_sb))

  def ring(idx_ref):
    gd = {0: pltpu.async_copy(x_hbm.at[idx_ref.at[0]], rows_a, sem_ga)}
    sd = {}
    for k in range(_CPP):
      buf, _, ssem = bufs[k % 2]
      nbuf, gsem_n, _ = bufs[(k + 1) % 2]
      if k + 1 < _CPP:
        if k - 1 >= 0:
          sd[k - 1].wait()
        gd[k + 1] = pltpu.async_copy(
            x_hbm.at[idx_ref.at[2 * (k + 1)]], nbuf, gsem_n)
      gd[k].wait()
      sd[k] = pltpu.async_copy(
          buf, agg_s.at[idx_ref.at[2 * k + 1]], ssem, add=True)
    sd[_CPP - 2].wait()
    sd[_CPP - 1].wait()

  pltpu.sync_copy(ei_hbm.at[wid, pl.ds(0, 2 * _CPP)], idx_a)
  plsc.subcore_barrier()

  def dbl_pass(q, carry):
    off_b = pl.multiple_of((2 * q + 1) * (2 * _CPP), 2 * _CPP)
    pfb = pltpu.async_copy(ei_hbm.at[wid, pl.ds(off_b, 2 * _CPP)],
                           idx_b, sem_ib)
    ring(idx_a)
    pfb.wait()
    off_a = pl.multiple_of(
        jnp.minimum(2 * q + 2, _NPASS - 1) * (2 * _CPP), 2 * _CPP)
    pfa = pltpu.async_copy(ei_hbm.at[wid, pl.ds(off_a, 2 * _CPP)],
                           idx_a, sem_ia)
    ring(idx_b)
    pfa.wait()
    return carry

  lax.fori_loop(0, _NPASS // 2, dbl_pass, 0)
  plsc.subcore_barrier()
  _write_out(agg_s, agg_out, c, s)


@functools.partial(
    pl.kernel,
    out_type=(jax.ShapeDtypeStruct((_NC, _N, _D), jnp.float32),),
    mesh=_mesh,
    scratch_types=[
        pltpu.VMEM((_CPP, _CH), jnp.int32),
        pltpu.VMEM((_CPP, _CH), jnp.int32),
        pltpu.VMEM((_CH, _D), jnp.float32),
        pltpu.VMEM_SHARED((_N, _D), jnp.float32),
        pltpu.SemaphoreType.DMA,
        pltpu.SemaphoreType.DMA,
        pltpu.SemaphoreType.DMA,
    ],
)
def _sc_deg(dst_hbm, z_hbm, ones_hbm, deg_out,
            dst_a, dst_b, ones_v, deg_s, sem, sem_ia, sem_ib):
  c = lax.axis_index("c")
  s = lax.axis_index("s")
  wid = c * _NS + s
  _init_zero(z_hbm, deg_s, s)
  pltpu.sync_copy(ones_hbm, ones_v)

  def burst(dst_ref):
    sd = [pltpu.async_copy(ones_v, deg_s.at[dst_ref.at[j]], sem, add=True)
          for j in range(_CPP)]
    for d in sd:
      d.wait()

  pltpu.sync_copy(dst_hbm.at[wid, pl.ds(0, _CPP)], dst_a)
  plsc.subcore_barrier()

  def dbl_pass(q, carry):
    off_b = pl.multiple_of((2 * q + 1) * _CPP, _CPP)
    pfb = pltpu.async_copy(dst_hbm.at[wid, pl.ds(off_b, _CPP)], dst_b, sem_ib)
    burst(dst_a)
    pfb.wait()
    off_a = pl.multiple_of(
        jnp.minimum(2 * q + 2, _NPASS - 1) * _CPP, _CPP)
    pfa = pltpu.async_copy(dst_hbm.at[wid, pl.ds(off_a, _CPP)], dst_a, sem_ia)
    burst(dst_b)
    pfa.wait()
    return carry

  lax.fori_loop(0, _NPASS // 2, dbl_pass, 0)
  plsc.subcore_barrier()
  _write_out(deg_s, deg_out, c, s)


_NB = 5
_BR = _N // _NB


def _mm2(a, wa, b, wb, bias):
  return (jnp.dot(a, wa, preferred_element_type=jnp.float32,
                  precision=lax.Precision.HIGHEST)
          + jnp.dot(b, wb, preferred_element_type=jnp.float32,
                    precision=lax.Precision.HIGHEST)
          + bias)


def _mean_agg(agg_ref, deg_ref):
  agg = agg_ref[0] + agg_ref[1]
  deg = deg_ref[0][:, 0] + deg_ref[1][:, 0]
  return agg / jnp.clip(deg, 1.0, None)[:, None]


def _tc1_body(agg_ref, deg_ref, x_ref, wl_ref, wr_ref, b_ref, g_ref,
              beta_ref, h_ref, hpre_s, sum_s, sq_s):
  i = pl.program_id(0)

  @pl.when(i < _NB)
  def _():
    mean = _mean_agg(agg_ref, deg_ref)
    h = _mm2(mean, wl_ref[...], x_ref[...], wr_ref[...], b_ref[...])
    hpre_s[i] = h

    @pl.when(i == 0)
    def _():
      sum_s[...] = jnp.zeros_like(sum_s)
      sq_s[...] = jnp.zeros_like(sq_s)

    sum_s[0, :] += jnp.sum(h, axis=0)
    sq_s[0, :] += jnp.sum(jnp.square(h), axis=0)

  @pl.when(i >= _NB)
  def _():
    mu = sum_s[0, :] * (1.0 / _N)
    var = sq_s[0, :] * (1.0 / _N) - jnp.square(mu)
    scale = g_ref[...] * lax.rsqrt(var + _EPS)
    hp = hpre_s[i - _NB]
    h_ref[...] = jnp.maximum((hp - mu) * scale + beta_ref[...], 0.0)


def _tc2_body(agg_ref, deg_ref, h_ref, wl_ref, wr_ref, b_ref, out_ref):
  mean = _mean_agg(agg_ref, deg_ref)
  out_ref[...] = _mm2(mean, wl_ref[...], h_ref[...], wr_ref[...], b_ref[...])


def _phase1_blk(i):
  return (jnp.where(i < _NB, i, 0), 0)


_blk_agg1 = pl.BlockSpec((_NC, _BR, _D), lambda i: (0,) + _phase1_blk(i))
_blk_row1 = pl.BlockSpec((_BR, _D), _phase1_blk)
_blk_vec1 = pl.BlockSpec((_D,), lambda i: (0,))
_blk_mat1 = pl.BlockSpec((_D, _D), lambda i: (0, 0))
_blk_out1 = pl.BlockSpec((_BR, _D), lambda i: (jnp.where(i < _NB, 0, i - _NB), 0))

_tc1 = pl.pallas_call(
    _tc1_body,
    grid=(2 * _NB,),
    in_specs=[_blk_agg1, _blk_agg1, _blk_row1, _blk_mat1, _blk_mat1,
              _blk_vec1, _blk_vec1, _blk_vec1],
    out_specs=_blk_out1,
    out_shape=jax.ShapeDtypeStruct((_N, _D), jnp.float32),
    scratch_shapes=[pltpu.VMEM((_NB, _BR, _D), jnp.float32),
                    pltpu.VMEM((1, _D), jnp.float32),
                    pltpu.VMEM((1, _D), jnp.float32)])

_blk_agg = pl.BlockSpec((_NC, _BR, _D), lambda i: (0, i, 0))
_blk_row = pl.BlockSpec((_BR, _D), lambda i: (i, 0))
_blk_vec = pl.BlockSpec((_D,), lambda i: (0,))
_blk_mat = pl.BlockSpec((_D, _D), lambda i: (0, 0))

_tc2 = pl.pallas_call(
    _tc2_body,
    grid=(_NB,),
    in_specs=[_blk_agg, _blk_agg, _blk_row, _blk_mat, _blk_mat, _blk_vec],
    out_specs=_blk_row,
    out_shape=jax.ShapeDtypeStruct((_N, _D), jnp.float32))


def kernel(x, edge_index, W1_l, W1_r, b1, gamma, beta, W2_l, W2_r, b2):
  src = edge_index[0].astype(jnp.int32).reshape(_NW, _NCHUNK, _CH)
  dst = edge_index[1].astype(jnp.int32).reshape(_NW, _NCHUNK, _CH)
  ei = jnp.stack([src, dst], axis=2).reshape(_NW, 2 * _NCHUNK, _CH)
  z = jnp.zeros((_N, _D), jnp.float32)
  ones = jnp.ones((_CH, _D), jnp.float32)

  (deg,) = _sc_deg(dst, z, ones)
  (agg1,) = _sc_agg(ei, x, z)
  h = _tc1(agg1, deg, x, W1_l, W1_r, b1, gamma, beta)
  (agg2,) = _sc_agg(ei, h, z)
  out = _tc2(agg2, deg, h, W2_l, W2_r, b2)
  return out


# speedup vs baseline: 1.0856x; 1.0194x over previous
import functools

import jax
import jax.numpy as jnp
from jax import lax
from jax.experimental import pallas as pl
from jax.experimental.pallas import tpu as pltpu
from jax.experimental.pallas import tpu_sc as plsc

_N = 10000
_D = 128
_E = 320000
_EPS = 1e-5

_NC = 2
_NS = 16
_NW = _NC * _NS
_EPT = _E // _NW
_CH = 125
_NCHUNK = _EPT // _CH
_NPASS = 10
_CPP = _NCHUNK // _NPASS
_RPT = 624
_TAIL0 = _RPT * _NS
_TAILN = _N - _TAIL0

_mesh = plsc.VectorSubcoreMesh(core_axis_name="c", subcore_axis_name="s")


def _init_zero(z_hbm, dst_s, s):
  row0 = s * _RPT
  pltpu.sync_copy(z_hbm.at[pl.ds(row0, _RPT)], dst_s.at[pl.ds(row0, _RPT)])

  @pl.when(s == _NS - 1)
  def _():
    pltpu.sync_copy(z_hbm.at[pl.ds(_TAIL0, _TAILN)],
                    dst_s.at[pl.ds(_TAIL0, _TAILN)])


def _write_out(src_s, out_hbm, c, s):
  row0 = s * _RPT
  pltpu.sync_copy(src_s.at[pl.ds(row0, _RPT)],
                  out_hbm.at[c, pl.ds(row0, _RPT)])

  @pl.when(s == _NS - 1)
  def _():
    pltpu.sync_copy(src_s.at[pl.ds(_TAIL0, _TAILN)],
                    out_hbm.at[c, pl.ds(_TAIL0, _TAILN)])


@functools.partial(
    pl.kernel,
    out_type=(jax.ShapeDtypeStruct((_NC, _N, _D), jnp.float32),),
    mesh=_mesh,
    scratch_types=[
        pltpu.VMEM((2 * _CPP, _CH), jnp.int32),
        pltpu.VMEM((2 * _CPP, _CH), jnp.int32),
        pltpu.VMEM((_CH, _D), jnp.float32),
        pltpu.VMEM((_CH, _D), jnp.float32),
        pltpu.VMEM_SHARED((_N, _D), jnp.float32),
        pltpu.SemaphoreType.DMA,
        pltpu.SemaphoreType.DMA,
        pltpu.SemaphoreType.DMA,
        pltpu.SemaphoreType.DMA,
        pltpu.SemaphoreType.DMA,
        pltpu.SemaphoreType.DMA,
    ],
)
def _sc_agg(ei_hbm, x_hbm, z_hbm, agg_out,
            idx_a, idx_b, rows_a, rows_b, agg_s,
            sem_ga, sem_gb, sem_sa, sem_sb, sem_ia, sem_ib):
  c = lax.axis_index("c")
  s = lax.axis_index("s")
  wid = c * _NS + s
  _init_zero(z_hbm, agg_s, s)

  bufs = ((rows_a, sem_ga, sem_sa), (rows_b, sem_gb, sem_sb))

  def ring(idx_ref):
    gd = {0: pltpu.async_copy(x_hbm.at[idx_ref.at[0]], rows_a, sem_ga)}
    sd = {}
    for k in range(_CPP):
      buf, _, ssem = bufs[k % 2]
      nbuf, gsem_n, _ = bufs[(k + 1) % 2]
      if k + 1 < _CPP:
        if k - 1 >= 0:
          sd[k - 1].wait()
        gd[k + 1] = pltpu.async_copy(
            x_hbm.at[idx_ref.at[2 * (k + 1)]], nbuf, gsem_n)
      gd[k].wait()
      sd[k] = pltpu.async_copy(
          buf, agg_s.at[idx_ref.at[2 * k + 1]], ssem, add=True)
    sd[_CPP - 2].wait()
    sd[_CPP - 1].wait()

  pltpu.sync_copy(ei_hbm.at[wid, pl.ds(0, 2 * _CPP)], idx_a)
  plsc.subcore_barrier()

  def dbl_pass(q, carry):
    off_b = pl.multiple_of((2 * q + 1) * (2 * _CPP), 2 * _CPP)
    pfb = pltpu.async_copy(ei_hbm.at[wid, pl.ds(off_b, 2 * _CPP)],
                           idx_b, sem_ib)
    ring(idx_a)
    pfb.wait()
    off_a = pl.multiple_of(
        jnp.minimum(2 * q + 2, _NPASS - 1) * (2 * _CPP), 2 * _CPP)
    pfa = pltpu.async_copy(ei_hbm.at[wid, pl.ds(off_a, 2 * _CPP)],
                           idx_a, sem_ia)
    ring(idx_b)
    pfa.wait()
    return carry

  lax.fori_loop(0, _NPASS // 2, dbl_pass, 0)
  plsc.subcore_barrier()
  _write_out(agg_s, agg_out, c, s)


@functools.partial(
    pl.kernel,
    out_type=(jax.ShapeDtypeStruct((_NC, _N, _D), jnp.float32),),
    mesh=_mesh,
    scratch_types=[
        pltpu.VMEM((_CPP, _CH), jnp.int32),
        pltpu.VMEM((_CPP, _CH), jnp.int32),
        pltpu.VMEM((_CH, _D), jnp.float32),
        pltpu.VMEM_SHARED((_N, _D), jnp.float32),
        pltpu.SemaphoreType.DMA,
        pltpu.SemaphoreType.DMA,
        pltpu.SemaphoreType.DMA,
    ],
)
def _sc_deg(dst_hbm, z_hbm, ones_hbm, deg_out,
            dst_a, dst_b, ones_v, deg_s, sem, sem_ia, sem_ib):
  c = lax.axis_index("c")
  s = lax.axis_index("s")
  wid = c * _NS + s
  _init_zero(z_hbm, deg_s, s)
  pltpu.sync_copy(ones_hbm, ones_v)

  def burst(dst_ref):
    sd = [pltpu.async_copy(ones_v, deg_s.at[dst_ref.at[j]], sem, add=True)
          for j in range(_CPP)]
    for d in sd:
      d.wait()

  pltpu.sync_copy(dst_hbm.at[wid, pl.ds(0, _CPP)], dst_a)
  plsc.subcore_barrier()

  def dbl_pass(q, carry):
    off_b = pl.multiple_of((2 * q + 1) * _CPP, _CPP)
    pfb = pltpu.async_copy(dst_hbm.at[wid, pl.ds(off_b, _CPP)], dst_b, sem_ib)
    burst(dst_a)
    pfb.wait()
    off_a = pl.multiple_of(
        jnp.minimum(2 * q + 2, _NPASS - 1) * _CPP, _CPP)
    pfa = pltpu.async_copy(dst_hbm.at[wid, pl.ds(off_a, _CPP)], dst_a, sem_ia)
    burst(dst_b)
    pfa.wait()
    return carry

  lax.fori_loop(0, _NPASS // 2, dbl_pass, 0)
  plsc.subcore_barrier()
  _write_out(deg_s, deg_out, c, s)


_NB = 2
_BR = _N // _NB


def _mm2(a, wa, b, wb, bias):
  return (jnp.dot(a, wa, preferred_element_type=jnp.float32)
          + jnp.dot(b, wb, preferred_element_type=jnp.float32)
          + bias)


def _mean_agg(agg_ref, deg_ref):
  agg = agg_ref[0] + agg_ref[1]
  deg = deg_ref[0][:, 0] + deg_ref[1][:, 0]
  return agg / jnp.clip(deg, 1.0, None)[:, None]


def _tc1_body(agg_ref, deg_ref, x_ref, wl_ref, wr_ref, b_ref, g_ref,
              beta_ref, h_ref, hpre_s, sum_s, sq_s):
  i = pl.program_id(0)

  @pl.when(i < _NB)
  def _():
    mean = _mean_agg(agg_ref, deg_ref)
    h = _mm2(mean, wl_ref[...], x_ref[...], wr_ref[...], b_ref[...])
    hpre_s[i] = h

    @pl.when(i == 0)
    def _():
      sum_s[...] = jnp.zeros_like(sum_s)
      sq_s[...] = jnp.zeros_like(sq_s)

    sum_s[0, :] += jnp.sum(h, axis=0)
    sq_s[0, :] += jnp.sum(jnp.square(h), axis=0)

  @pl.when(i >= _NB)
  def _():
    mu = sum_s[0, :] * (1.0 / _N)
    var = sq_s[0, :] * (1.0 / _N) - jnp.square(mu)
    scale = g_ref[...] * lax.rsqrt(var + _EPS)
    hp = hpre_s[i - _NB]
    h_ref[...] = jnp.maximum((hp - mu) * scale + beta_ref[...], 0.0)


def _tc2_body(agg_ref, deg_ref, h_ref, wl_ref, wr_ref, b_ref, out_ref):
  mean = _mean_agg(agg_ref, deg_ref)
  out_ref[...] = _mm2(mean, wl_ref[...], h_ref[...], wr_ref[...], b_ref[...])


def _phase1_blk(i):
  return (jnp.where(i < _NB, i, 0), 0)


_blk_agg1 = pl.BlockSpec((_NC, _BR, _D), lambda i: (0,) + _phase1_blk(i))
_blk_row1 = pl.BlockSpec((_BR, _D), _phase1_blk)
_blk_vec1 = pl.BlockSpec((_D,), lambda i: (0,))
_blk_mat1 = pl.BlockSpec((_D, _D), lambda i: (0, 0))
_blk_out1 = pl.BlockSpec((_BR, _D), lambda i: (jnp.where(i < _NB, 0, i - _NB), 0))

_tc1 = pl.pallas_call(
    _tc1_body,
    grid=(2 * _NB,),
    in_specs=[_blk_agg1, _blk_agg1, _blk_row1, _blk_mat1, _blk_mat1,
              _blk_vec1, _blk_vec1, _blk_vec1],
    out_specs=_blk_out1,
    out_shape=jax.ShapeDtypeStruct((_N, _D), jnp.float32),
    scratch_shapes=[pltpu.VMEM((_NB, _BR, _D), jnp.float32),
                    pltpu.VMEM((1, _D), jnp.float32),
                    pltpu.VMEM((1, _D), jnp.float32)])

_blk_agg = pl.BlockSpec((_NC, _BR, _D), lambda i: (0, i, 0))
_blk_row = pl.BlockSpec((_BR, _D), lambda i: (i, 0))
_blk_vec = pl.BlockSpec((_D,), lambda i: (0,))
_blk_mat = pl.BlockSpec((_D, _D), lambda i: (0, 0))

_tc2 = pl.pallas_call(
    _tc2_body,
    grid=(_NB,),
    in_specs=[_blk_agg, _blk_agg, _blk_row, _blk_mat, _blk_mat, _blk_vec],
    out_specs=_blk_row,
    out_shape=jax.ShapeDtypeStruct((_N, _D), jnp.float32))


def kernel(x, edge_index, W1_l, W1_r, b1, gamma, beta, W2_l, W2_r, b2):
  src = edge_index[0].astype(jnp.int32).reshape(_NW, _NCHUNK, _CH)
  dst = edge_index[1].astype(jnp.int32).reshape(_NW, _NCHUNK, _CH)
  ei = jnp.stack([src, dst], axis=2).reshape(_NW, 2 * _NCHUNK, _CH)
  z = jnp.zeros((_N, _D), jnp.float32)
  ones = jnp.ones((_CH, _D), jnp.float32)

  (deg,) = _sc_deg(dst, z, ones)
  (agg1,) = _sc_agg(ei, x, z)
  h = _tc1(agg1, deg, x, W1_l, W1_r, b1, gamma, beta)
  (agg2,) = _sc_agg(ei, h, z)
  out = _tc2(agg2, deg, h, W2_l, W2_r, b2)
  return out
